# baseline (speedup 1.0000x reference)
import functools

import numpy as np
import jax
import jax.numpy as jnp
from jax import lax
from jax.experimental import pallas as pl
from jax.experimental.pallas import tpu as pltpu
from jax.experimental.pallas import tpu_sc as plsc

_B, _A, _H, _W = 8, 3, 40, 40
_EPS = 1e-7
_ANCH = np.array([[0.05, 0.06], [0.12, 0.15], [0.30, 0.35]], np.float32)
_AW = [float(_ANCH[a, 0] * _W) for a in range(_A)]
_AH = [float(_ANCH[a, 1] * _H) for a in range(_A)]
_W2H2 = [float(np.float32(_AW[a]) * (np.float32(_AH[a]) + np.float32(_EPS))) for a in range(_A)]
_QX = (0, 1, 0, 1)
_QY = (0, 0, 1, 1)

_NT = 32
_NPT = 640
_NPAD = _NT * _NPT
_NG = _NPT // 16
_TABW = _B * _A * _H * 128
_PB = _A * _H * 128
_PA = _H * 128

_FIX = 32767.0


def _tc_prep(x_ref, o_ref):
    zpad = jnp.zeros((_H, 24), jnp.int32)
    for a in range(_A):
        t0 = jnp.tanh(x_ref[0, a, 0])
        t1 = jnp.tanh(x_ref[0, a, 1])
        i0 = (t0 * _FIX + jnp.where(t0 >= 0, 0.5, -0.5)).astype(jnp.int32)
        i1 = (t1 * _FIX + jnp.where(t1 >= 0, 0.5, -0.5)).astype(jnp.int32)
        w0 = (i0 & 0xFFFF) | (i1 << 16)
        e0 = jnp.exp(x_ref[0, a, 2]) * _AW[a]
        e1 = jnp.exp(x_ref[0, a, 3]) * _AH[a]
        b0 = lax.bitcast_convert_type(e0.astype(jnp.bfloat16), jnp.uint16).astype(jnp.int32)
        b1 = lax.bitcast_convert_type(e1.astype(jnp.bfloat16), jnp.uint16).astype(jnp.int32)
        w1 = b0 | (b1 << 16)
        plane = jnp.concatenate([w0, zpad, w1, zpad], axis=1)
        o_ref[pl.ds(a * _PA, _PA)] = plane.reshape(_PA)


def _sc_body(tg, ptab, out, tv, tab, mbits, ostage, sem, nvalid, chunk):
    wid = lax.axis_index("s") * 2 + lax.axis_index("c")
    cp_tab = pltpu.async_copy(ptab, tab, sem)
    start = jnp.minimum(wid * chunk, nvalid - chunk)
    pltpu.sync_copy(tg.at[pl.ds(start, chunk)], tv.at[pl.ds(0, chunk)])
    lo = wid * chunk - start
    iota = lax.iota(jnp.int32, 16)

    def load_gt(s):
        ti = s + iota
        f2 = jnp.full((16,), 2, jnp.int32)
        gx = plsc.load_gather(tv, [ti, f2]) * 40.0
        gy = plsc.load_gather(tv, [ti, f2 + 1]) * 40.0
        gw = plsc.load_gather(tv, [ti, f2 + 2]) * 40.0
        gh = plsc.load_gather(tv, [ti, f2 + 3]) * 40.0
        cx = gx.astype(jnp.int32)
        cy = gy.astype(jnp.int32)
        gx1 = gx - gw * 0.5
        gx2 = gx + gw * 0.5
        gy1 = gy - gh * 0.5
        gy2 = gy + gh * 0.5
        return cx, cy, gx1, gx2, gy1, gy2

    def loop1(g, carry):
        s = g * 16
        cx, cy, gx1, gx2, gy1, gy2 = load_gt(s)
        w1h1 = (gx2 - gx1) * (gy2 - gy1 + _EPS)
        ious = []
        inbs = []
        for q in range(4):
            pxi = cx + _QX[q]
            pyi = cy + _QY[q]
            fx = pxi.astype(jnp.float32)
            fy = pyi.astype(jnp.float32)
            inb = (pxi >= 0) & (pxi < _W) & (pyi >= 0) & (pyi < _H)
            for a in range(_A):
                hw = _AW[a] * 0.5
                hh = _AH[a] * 0.5
                iw = jnp.maximum(jnp.minimum(gx2, fx + hw) - jnp.maximum(gx1, fx - hw), 0.0)
                ih = jnp.maximum(jnp.minimum(gy2, fy + hh) - jnp.maximum(gy1, fy - hh), 0.0)
                inter = iw * ih
                union = w1h1 + _W2H2[a] - inter + _EPS
                ious.append(inter / union)
                inbs.append(inb)
        best = ious[0]
        bidx = jnp.zeros((16,), jnp.int32)
        any07 = ious[0] > 0.7
        for k in range(1, 12):
            gtm = ious[k] > best
            best = jnp.where(gtm, ious[k], best)
            bidx = jnp.where(gtm, jnp.int32(k), bidx)
            any07 = any07 | (ious[k] > 0.7)
        none07 = ~any07
        mb = jnp.zeros((16,), jnp.int32)
        for k in range(12):
            m = (ious[k] > 0.7) | (none07 & (bidx == k))
            m = m & inbs[k]
            mb = mb | jnp.where(m, jnp.int32(1 << k), jnp.int32(0))
        local = s + iota
        valid = (local >= lo) & (local < chunk)
        mbits[pl.ds(s, 16)] = jnp.where(valid, mb, 0)
        return carry

    with jax.named_scope("mask_phase"):
        lax.fori_loop(0, _NG, loop1, jnp.int32(0))

    with jax.named_scope("tab_wait"):
        cp_tab.wait()

    def loop2(g, carry):
        acc, cntv = carry
        s = g * 16
        cx, cy, gx1, gx2, gy1, gy2 = load_gt(s)
        bi = plsc.load_gather(tv, [s + iota, jnp.full((16,), 0, jnp.int32)])
        bi = jnp.clip(bi.astype(jnp.int32), 0, _B - 1)
        boff = bi * _PB
        g_sumx = gx1 + gx2
        g_sumy = gy1 + gy2
        w2 = gx2 - gx1
        h2 = gy2 - gy1 + _EPS
        w2h2 = w2 * h2
        mb = mbits[pl.ds(s, 16)]
        for q in range(4):
            pxi = cx + _QX[q]
            pyi = cy + _QY[q]
            fx = pxi.astype(jnp.float32)
            fy = pyi.astype(jnp.float32)
            xs = jnp.clip(pxi, 0, _W - 1)
            ys = jnp.clip(pyi, 0, _H - 1)
            pq = boff + (ys << 7) + xs
            for a in range(_A):
                k = q * 3 + a
                w0 = plsc.load_gather(tab, [pq + (a * _PA)])
                w1 = plsc.load_gather(tab, [pq + (a * _PA + 64)])
                sel0 = ((w0 << 16) >> 16).astype(jnp.float32) * (1.0 / _FIX)
                sel1 = (w0 >> 16).astype(jnp.float32) * (1.0 / _FIX)
                pbw = plsc.bitcast(w1 << 16, jnp.float32)
                pbh = plsc.bitcast(w1 & jnp.int32(-65536), jnp.float32)
                pbx = sel0 + fx
                pby = sel1 + fy
                bx1 = pbx - pbw * 0.5
                bx2 = pbx + pbw * 0.5
                by1 = pby - pbh * 0.5
                by2 = pby + pbh * 0.5
                w1s = bx2 - bx1
                h1s = by2 - by1 + _EPS
                iw = jnp.maximum(jnp.minimum(bx2, gx2) - jnp.maximum(bx1, gx1), 0.0)
                ih = jnp.maximum(jnp.minimum(by2, gy2) - jnp.maximum(by1, gy1), 0.0)
                inter = iw * ih
                union = w1s * h1s + w2h2 - inter + _EPS
                iou = inter / union
                scw = (g_sumx - bx1 - bx2) * 0.5
                sch = (g_sumy - by1 - by2) * 0.5
                den = scw * scw + sch * sch
                acost = 2.0 * jnp.abs(scw) * jnp.abs(sch) / den
                cw = jnp.maximum(bx2, gx2) - jnp.minimum(bx1, gx1)
                chh = jnp.maximum(by2, gy2) - jnp.minimum(by1, gy1)
                rx = scw / cw
                rx = rx * rx
                ry = sch / chh
                ry = ry * ry
                gam = acost - 2.0
                dcost = 2.0 - jnp.exp(gam * rx) - jnp.exp(gam * ry)
                ow = jnp.abs(w1s - w2) / jnp.maximum(w1s, w2)
                oh = jnp.abs(h1s - h2) / jnp.maximum(h1s, h2)
                ew = 1.0 - jnp.exp(-ow)
                ew = ew * ew
                ew = ew * ew
                eh = 1.0 - jnp.exp(-oh)
                eh = eh * eh
                eh = eh * eh
                siou = iou - 0.5 * (dcost + ew + eh)
                mk = ((mb >> k) & 1) > 0
                acc = acc + jnp.where(mk, 1.0 - siou, 0.0)
                cntv = cntv + jnp.where(mk, 1.0, 0.0)
        return acc, cntv

    with jax.named_scope("siou_phase"):
        acc, cntv = lax.fori_loop(
            0, _NG, loop2,
            (jnp.zeros((16,), jnp.float32), jnp.zeros((16,), jnp.float32)),
        )
    ostage[0, :] = acc
    ostage[1, :] = cntv
    pltpu.sync_copy(ostage, out.at[wid])


@jax.jit
def _run(pred_delta_box, targets):
    n = targets.shape[0]
    chunk = -(-n // _NT)
    chunk = (chunk + 3) // 4 * 4
    ptab = pl.pallas_call(
        _tc_prep,
        grid=(_B,),
        in_specs=[pl.BlockSpec((1, _A, 4, _H, _W), lambda b: (b, 0, 0, 0, 0))],
        out_specs=pl.BlockSpec((_PB,), lambda b: (b,)),
        out_shape=jax.ShapeDtypeStruct((_TABW,), jnp.int32),
    )(pred_delta_box)

    mesh = plsc.VectorSubcoreMesh(core_axis_name="c", subcore_axis_name="s")
    sc = functools.partial(
        pl.kernel,
        mesh=mesh,
        compiler_params=pltpu.CompilerParams(
            needs_layout_passes=False, use_tc_tiling_on_sc=False
        ),
        out_type=jax.ShapeDtypeStruct((_NT, 2, 16), jnp.float32),
        scratch_types=[
            pltpu.VMEM((_NPT, 6), jnp.float32),
            pltpu.VMEM((_TABW,), jnp.int32),
            pltpu.VMEM((_NPT,), jnp.int32),
            pltpu.VMEM((2, 16), jnp.float32),
            pltpu.SemaphoreType.DMA,
        ],
    )(functools.partial(_sc_body, nvalid=n, chunk=chunk))
    parts = sc(targets, ptab)
    return jnp.sum(parts[:, 0, :]) / jnp.sum(parts[:, 1, :])


def kernel(pred_obj, pred_delta_box, pred_cls, targets):
    return _run(pred_delta_box, targets)


# speedup vs baseline: 1.2375x; 1.2375x over previous
import functools

import numpy as np
import jax
import jax.numpy as jnp
from jax import lax
from jax.experimental import pallas as pl
from jax.experimental.pallas import tpu as pltpu
from jax.experimental.pallas import tpu_sc as plsc

_B, _A, _H, _W = 8, 3, 40, 40
_EPS = 1e-7
_ANCH = np.array([[0.05, 0.06], [0.12, 0.15], [0.30, 0.35]], np.float32)
_AW = [float(_ANCH[a, 0] * _W) for a in range(_A)]
_AH = [float(_ANCH[a, 1] * _H) for a in range(_A)]
_W2H2 = [float(np.float32(_AW[a]) * (np.float32(_AH[a]) + np.float32(_EPS))) for a in range(_A)]
_QX = (0, 1, 0, 1)
_QY = (0, 0, 1, 1)

_NT = 32
_NPT = 640
_NPAD = _NT * _NPT
_NG = _NPT // 16
_HW = _H * _W
_PA = 2 * _HW
_PB = _A * _PA
_TABW = _B * _PB

_FIX = 32767.0


def _tc_prep(x_ref, t_ref, o_ref, s_ref):
    for bb in range(_B):
        for a in range(_A):
            r = bb * 12 + 4 * a
            t0 = jnp.tanh(x_ref[r + 0, :])
            t1 = jnp.tanh(x_ref[r + 1, :])
            i0 = (t0 * _FIX + jnp.where(t0 >= 0, 0.5, -0.5)).astype(jnp.int32)
            i1 = (t1 * _FIX + jnp.where(t1 >= 0, 0.5, -0.5)).astype(jnp.int32)
            o_ref[pl.ds(bb * _PB + a * _PA, _HW)] = (i0 & 0xFFFF) | (i1 << 16)
            e0 = jnp.exp(x_ref[r + 2, :]) * _AW[a]
            e1 = jnp.exp(x_ref[r + 3, :]) * _AH[a]
            b0 = lax.bitcast_convert_type(e0.astype(jnp.bfloat16), jnp.uint16).astype(jnp.int32)
            b1 = lax.bitcast_convert_type(e1.astype(jnp.bfloat16), jnp.uint16).astype(jnp.int32)
            o_ref[pl.ds(bb * _PB + a * _PA + _HW, _HW)] = b0 | (b1 << 16)
    tr = jnp.transpose(t_ref[...])
    rid = lax.broadcasted_iota(jnp.int32, tr.shape, 0)
    tr = jnp.where(rid < 2, tr, tr * 40.0)
    tr = jnp.concatenate(
        [tr, jnp.zeros((6, _NPAD - tr.shape[1]), jnp.float32)], axis=1
    )
    zpad2 = jnp.zeros((2, 128), jnp.float32)
    for cb in range(_NPAD // 128):
        slab = jnp.concatenate([tr[:, cb * 128:(cb + 1) * 128], zpad2], axis=0)
        s_ref[pl.ds(cb * 1024, 1024)] = slab.reshape(1024)


def _sc_body(tsoa, ptab, out, tv, tab, mbits, ostage, sem, nvalid):
    wid = lax.axis_index("s") * 2 + lax.axis_index("c")
    base = wid * _NPT
    cp_tab = pltpu.async_copy(ptab, tab, sem)
    pltpu.sync_copy(tsoa.at[pl.ds(wid * 5120, 5120)], tv)
    iota = lax.iota(jnp.int32, 16)

    def load_gt(cb, off):
        o = cb * 1024 + off
        gx = tv[pl.ds(o + 2 * 128, 16)]
        gy = tv[pl.ds(o + 3 * 128, 16)]
        gw = tv[pl.ds(o + 4 * 128, 16)]
        gh = tv[pl.ds(o + 5 * 128, 16)]
        cx = gx.astype(jnp.int32)
        cy = gy.astype(jnp.int32)
        gx1 = gx - gw * 0.5
        gx2 = gx + gw * 0.5
        gy1 = gy - gh * 0.5
        gy2 = gy + gh * 0.5
        return cx, cy, gx1, gx2, gy1, gy2

    def loop1(g, carry):
        s = g * 16
        cb = g >> 3
        off = (g & 7) * 16
        cx, cy, gx1, gx2, gy1, gy2 = load_gt(cb, off)
        w1h1 = (gx2 - gx1) * (gy2 - gy1 + _EPS)
        ious = []
        inbs = []
        for q in range(4):
            pxi = cx + _QX[q]
            pyi = cy + _QY[q]
            fx = pxi.astype(jnp.float32)
            fy = pyi.astype(jnp.float32)
            inb = (pxi >= 0) & (pxi < _W) & (pyi >= 0) & (pyi < _H)
            for a in range(_A):
                hw = _AW[a] * 0.5
                hh = _AH[a] * 0.5
                iw = jnp.maximum(jnp.minimum(gx2, fx + hw) - jnp.maximum(gx1, fx - hw), 0.0)
                ih = jnp.maximum(jnp.minimum(gy2, fy + hh) - jnp.maximum(gy1, fy - hh), 0.0)
                inter = iw * ih
                union = w1h1 + _W2H2[a] - inter + _EPS
                ious.append(inter / union)
                inbs.append(inb)
        best = ious[0]
        bidx = jnp.zeros((16,), jnp.int32)
        any07 = ious[0] > 0.7
        for k in range(1, 12):
            gtm = ious[k] > best
            best = jnp.where(gtm, ious[k], best)
            bidx = jnp.where(gtm, jnp.int32(k), bidx)
            any07 = any07 | (ious[k] > 0.7)
        none07 = ~any07
        mb = jnp.zeros((16,), jnp.int32)
        for k in range(12):
            m = (ious[k] > 0.7) | (none07 & (bidx == k))
            m = m & inbs[k]
            mb = mb | jnp.where(m, jnp.int32(1 << k), jnp.int32(0))
        valid = (base + s + iota) < nvalid
        mbits[pl.ds(s, 16)] = jnp.where(valid, mb, 0)
        return carry

    with jax.named_scope("mask_phase"):
        lax.fori_loop(0, _NG, loop1, jnp.int32(0))

    with jax.named_scope("tab_wait"):
        cp_tab.wait()

    def loop2(g, carry):
        acc, cntv = carry
        s = g * 16
        cb = g >> 3
        off = (g & 7) * 16
        cx, cy, gx1, gx2, gy1, gy2 = load_gt(cb, off)
        bi = jnp.clip(tv[pl.ds(cb * 1024 + off, 16)].astype(jnp.int32), 0, _B - 1)
        boff = bi * _PB
        g_sumx = gx1 + gx2
        g_sumy = gy1 + gy2
        w2 = gx2 - gx1
        h2 = gy2 - gy1 + _EPS
        w2h2 = w2 * h2
        mb = mbits[pl.ds(s, 16)]
        for q in range(4):
            pxi = cx + _QX[q]
            pyi = cy + _QY[q]
            fx = pxi.astype(jnp.float32)
            fy = pyi.astype(jnp.float32)
            xs = jnp.clip(pxi, 0, _W - 1)
            ys = jnp.clip(pyi, 0, _H - 1)
            pq = boff + ys * _W + xs
            for a in range(_A):
                k = q * 3 + a
                w0 = plsc.load_gather(tab, [pq + (a * _PA)])
                w1 = plsc.load_gather(tab, [pq + (a * _PA + _HW)])
                sel0 = ((w0 << 16) >> 16).astype(jnp.float32) * (1.0 / _FIX)
                sel1 = (w0 >> 16).astype(jnp.float32) * (1.0 / _FIX)
                pbw = plsc.bitcast(w1 << 16, jnp.float32)
                pbh = plsc.bitcast(w1 & jnp.int32(-65536), jnp.float32)
                pbx = sel0 + fx
                pby = sel1 + fy
                bx1 = pbx - pbw * 0.5
                bx2 = pbx + pbw * 0.5
                by1 = pby - pbh * 0.5
                by2 = pby + pbh * 0.5
                w1s = bx2 - bx1
                h1s = by2 - by1 + _EPS
                iw = jnp.maximum(jnp.minimum(bx2, gx2) - jnp.maximum(bx1, gx1), 0.0)
                ih = jnp.maximum(jnp.minimum(by2, gy2) - jnp.maximum(by1, gy1), 0.0)
                inter = iw * ih
                union = w1s * h1s + w2h2 - inter + _EPS
                iou = inter / union
                scw = (g_sumx - bx1 - bx2) * 0.5
                sch = (g_sumy - by1 - by2) * 0.5
                den = scw * scw + sch * sch
                acost = 2.0 * jnp.abs(scw) * jnp.abs(sch) / den
                cw = jnp.maximum(bx2, gx2) - jnp.minimum(bx1, gx1)
                chh = jnp.maximum(by2, gy2) - jnp.minimum(by1, gy1)
                rx = scw / cw
                rx = rx * rx
                ry = sch / chh
                ry = ry * ry
                gam = acost - 2.0
                dcost = 2.0 - jnp.exp(gam * rx) - jnp.exp(gam * ry)
                ow = jnp.abs(w1s - w2) / jnp.maximum(w1s, w2)
                oh = jnp.abs(h1s - h2) / jnp.maximum(h1s, h2)
                ew = 1.0 - jnp.exp(-ow)
                ew = ew * ew
                ew = ew * ew
                eh = 1.0 - jnp.exp(-oh)
                eh = eh * eh
                eh = eh * eh
                siou = iou - 0.5 * (dcost + ew + eh)
                mk = ((mb >> k) & 1) > 0
                acc = acc + jnp.where(mk, 1.0 - siou, 0.0)
                cntv = cntv + jnp.where(mk, 1.0, 0.0)
        return acc, cntv

    with jax.named_scope("siou_phase"):
        acc, cntv = lax.fori_loop(
            0, _NG, loop2,
            (jnp.zeros((16,), jnp.float32), jnp.zeros((16,), jnp.float32)),
        )
    ostage[0, :] = acc
    ostage[1, :] = cntv
    pltpu.sync_copy(ostage, out.at[wid])


@jax.jit
def _run(pred_delta_box, targets):
    n = targets.shape[0]
    pdb2 = pred_delta_box.reshape(_B * _A * 4, _HW)
    ptab, tsoa = pl.pallas_call(
        _tc_prep,
        out_shape=[
            jax.ShapeDtypeStruct((_TABW,), jnp.int32),
            jax.ShapeDtypeStruct((_NT * 5120,), jnp.float32),
        ],
    )(pdb2, targets)

    mesh = plsc.VectorSubcoreMesh(core_axis_name="c", subcore_axis_name="s")
    sc = functools.partial(
        pl.kernel,
        mesh=mesh,
        compiler_params=pltpu.CompilerParams(
            needs_layout_passes=False, use_tc_tiling_on_sc=False
        ),
        out_type=jax.ShapeDtypeStruct((_NT, 2, 16), jnp.float32),
        scratch_types=[
            pltpu.VMEM((5120,), jnp.float32),
            pltpu.VMEM((_TABW,), jnp.int32),
            pltpu.VMEM((_NPT,), jnp.int32),
            pltpu.VMEM((2, 16), jnp.float32),
            pltpu.SemaphoreType.DMA,
        ],
    )(functools.partial(_sc_body, nvalid=n))
    parts = sc(tsoa, ptab)
    return jnp.sum(parts[:, 0, :]) / jnp.sum(parts[:, 1, :])


def kernel(pred_obj, pred_delta_box, pred_cls, targets):
    return _run(pred_delta_box, targets)
